# Initial kernel scaffold; baseline (speedup 1.0000x reference)
import jax
import jax.numpy as jnp
from jax.experimental import pallas as pl


def kernel(x, weight, bias, Wq, bq, Wk, bk, Wa, ba):
    raise NotImplementedError("write your pallas kernel here")


# speedup vs baseline: 17.8610x; 17.8610x over previous
import jax
import jax.numpy as jnp
from jax import lax
from jax.experimental import pallas as pl

NUM_LEVELS = 3
THRESHOLD = 0.1


def _ge_kernel(h_ref, weight_ref, bias_ref, wq_ref, bq_ref, wk_ref, bk_ref,
               wa_ref, ba_ref, out_ref):
    h = h_ref[0]
    n = h.shape[0]
    d = h.shape[1]
    weight = weight_ref[...]
    bias = bias_ref[...]
    wa = wa_ref[...]
    wa_q = wa[:, :d]
    wa_k = wa[:, d:]
    vq = jnp.dot(wa_q, wq_ref[...], preferred_element_type=jnp.float32)
    vk = jnp.dot(wa_k, wk_ref[...], preferred_element_type=jnp.float32)
    cq = jnp.sum(bq_ref[...] * wa_q)
    ck = jnp.sum(bk_ref[...] * wa_k)
    const = cq + ck + ba_ref[0, 0]

    row = lax.broadcasted_iota(jnp.int32, (n, n), 0)
    col = lax.broadcasted_iota(jnp.int32, (n, n), 1)
    offdiag = row != col

    for _ in range(NUM_LEVELS):
        sq = jnp.dot(h, vq.T, preferred_element_type=jnp.float32)
        sk = jnp.dot(h, vk.T, preferred_element_type=jnp.float32)
        scores = sq + sk.T + const
        probs = jax.nn.sigmoid(scores)
        w_edge = jnp.where(offdiag & (probs > THRESHOLD), probs, 0.0)
        deg = jnp.sum(w_edge, axis=0, keepdims=True)
        dinv = jnp.where(deg > 0, lax.rsqrt(deg), 0.0)
        w_norm = dinv.T * w_edge * dinv
        xw = jnp.dot(h, weight, preferred_element_type=jnp.float32)
        h = lax.dot_general(w_norm, xw, (((0,), (0,)), ((), ())),
                            preferred_element_type=jnp.float32) + bias
    out_ref[0] = h


def kernel(x, weight, bias, Wq, bq, Wk, bk, Wa, ba):
    b, d, n = x.shape[0], x.shape[1], x.shape[2]
    h = jnp.transpose(x, (0, 2, 1))
    bias2 = bias.reshape(1, d)
    bq2 = bq.reshape(1, d)
    bk2 = bk.reshape(1, d)
    ba2 = ba.reshape(1, 1)
    out = pl.pallas_call(
        _ge_kernel,
        grid=(b,),
        in_specs=[
            pl.BlockSpec((1, n, d), lambda i: (i, 0, 0)),
            pl.BlockSpec((d, d), lambda i: (0, 0)),
            pl.BlockSpec((1, d), lambda i: (0, 0)),
            pl.BlockSpec((d, d), lambda i: (0, 0)),
            pl.BlockSpec((1, d), lambda i: (0, 0)),
            pl.BlockSpec((d, d), lambda i: (0, 0)),
            pl.BlockSpec((1, d), lambda i: (0, 0)),
            pl.BlockSpec((1, 2 * d), lambda i: (0, 0)),
            pl.BlockSpec((1, 1), lambda i: (0, 0)),
        ],
        out_specs=pl.BlockSpec((1, n, d), lambda i: (i, 0, 0)),
        out_shape=jax.ShapeDtypeStruct((b, n, d), jnp.float32),
    )(h, weight, bias2, Wq, bq2, Wk, bk2, Wa, ba2)
    return jnp.transpose(out, (0, 2, 1))


# speedup vs baseline: 26.4980x; 1.4836x over previous
import jax
import jax.numpy as jnp
from jax import lax
from jax.experimental import pallas as pl

NUM_LEVELS = 3
THRESHOLD = 0.1


def _ge_kernel(h_ref, weight_ref, bias_ref, wq_ref, bq_ref, wk_ref, bk_ref,
               wa_ref, ba_ref, out_ref):
    b, n, d = h_ref.shape
    weight = weight_ref[...]
    bias = bias_ref[...]
    wa = wa_ref[...]
    wa_q = wa[:, :d]
    wa_k = wa[:, d:]
    vq = jnp.dot(wa_q, wq_ref[...], preferred_element_type=jnp.float32)
    vk = jnp.dot(wa_k, wk_ref[...], preferred_element_type=jnp.float32)
    cq = jnp.sum(bq_ref[...] * wa_q)
    ck = jnp.sum(bk_ref[...] * wa_k)
    const = cq + ck + ba_ref[0, 0]

    row = lax.broadcasted_iota(jnp.int32, (n, n), 0)
    col = lax.broadcasted_iota(jnp.int32, (n, n), 1)
    offdiag = row != col

    hs = [h_ref[i] for i in range(b)]
    for _ in range(NUM_LEVELS):
        new_hs = []
        for i in range(b):
            h = hs[i]
            sq = jnp.dot(h, vq.T, preferred_element_type=jnp.float32)
            sk = jnp.dot(h, vk.T, preferred_element_type=jnp.float32)
            scores = sq + sk.T + const
            probs = jax.nn.sigmoid(scores)
            w_edge = jnp.where(offdiag & (probs > THRESHOLD), probs, 0.0)
            deg = jnp.sum(w_edge, axis=0, keepdims=True)
            dinv = jnp.where(deg > 0, lax.rsqrt(deg), 0.0)
            w_norm = dinv.T * w_edge * dinv
            xw = jnp.dot(h, weight, preferred_element_type=jnp.float32)
            new_hs.append(
                lax.dot_general(w_norm, xw, (((0,), (0,)), ((), ())),
                                preferred_element_type=jnp.float32) + bias)
        hs = new_hs
    for i in range(b):
        out_ref[i] = hs[i]


def kernel(x, weight, bias, Wq, bq, Wk, bk, Wa, ba):
    b, d, n = x.shape[0], x.shape[1], x.shape[2]
    h = jnp.transpose(x, (0, 2, 1))
    bias2 = bias.reshape(1, d)
    bq2 = bq.reshape(1, d)
    bk2 = bk.reshape(1, d)
    ba2 = ba.reshape(1, 1)
    out = pl.pallas_call(
        _ge_kernel,
        out_shape=jax.ShapeDtypeStruct((b, n, d), jnp.float32),
    )(h, weight, bias2, Wq, bq2, Wk, bk2, Wa, ba2)
    return jnp.transpose(out, (0, 2, 1))


# speedup vs baseline: 36.3900x; 1.3733x over previous
import jax
import jax.numpy as jnp
from jax import lax
from jax.experimental import pallas as pl

NUM_LEVELS = 3
THRESHOLD = 0.1


def _ge_kernel(h_ref, weight_ref, bias_ref, wq_ref, bq_ref, wk_ref, bk_ref,
               wa_ref, ba_ref, out_ref):
    b, n, d = h_ref.shape
    bias = bias_ref[...]
    wa = wa_ref[...]
    wa_q = wa[:, :d]
    wa_k = wa[:, d:]
    vq = jnp.dot(wa_q, wq_ref[...], preferred_element_type=jnp.float32)
    vk = jnp.dot(wa_k, wk_ref[...], preferred_element_type=jnp.float32)
    cq = jnp.sum(bq_ref[...] * wa_q)
    ck = jnp.sum(bk_ref[...] * wa_k)
    const = cq + ck + ba_ref[0, 0]
    w_ext = jnp.concatenate([weight_ref[...], vq.T, vk.T], axis=1)
    ones_col = jnp.ones((n, 1), dtype=jnp.float32)

    row = lax.broadcasted_iota(jnp.int32, (1, n, n), 1)
    col = lax.broadcasted_iota(jnp.int32, (1, n, n), 2)
    offdiag = row != col

    hs = [h_ref[i] for i in range(b)]
    for _ in range(NUM_LEVELS):
        hws = [jnp.dot(h, w_ext, preferred_element_type=jnp.float32)
               for h in hs]
        xws = [hw[:, :d] for hw in hws]
        ss = jnp.stack([hw[:, d:d + 2] for hw in hws])
        sq_col = ss[:, :, 0:1]
        sk_col = ss[:, :, 1:2]
        sq_row = jnp.transpose(sq_col, (0, 2, 1))
        scores = sk_col + sq_row + const
        probs = jax.nn.sigmoid(scores)
        a_edge = jnp.where(offdiag & (probs > THRESHOLD), probs, 0.0)
        deg = jnp.stack([jnp.dot(a_edge[i], ones_col,
                                 preferred_element_type=jnp.float32)
                         for i in range(b)])
        dinv_col = jnp.where(deg > 0, lax.rsqrt(deg), 0.0)
        dinv_row = jnp.transpose(dinv_col, (0, 2, 1))
        a_norm = dinv_col * a_edge * dinv_row
        hs = [jnp.dot(a_norm[i], xws[i],
                      preferred_element_type=jnp.float32) + bias
              for i in range(b)]
    for i in range(b):
        out_ref[i] = hs[i]


def kernel(x, weight, bias, Wq, bq, Wk, bk, Wa, ba):
    b, d, n = x.shape[0], x.shape[1], x.shape[2]
    h = jnp.transpose(x, (0, 2, 1))
    bias2 = bias.reshape(1, d)
    bq2 = bq.reshape(1, d)
    bk2 = bk.reshape(1, d)
    ba2 = ba.reshape(1, 1)
    out = pl.pallas_call(
        _ge_kernel,
        out_shape=jax.ShapeDtypeStruct((b, n, d), jnp.float32),
    )(h, weight, bias2, Wq, bq2, Wk, bk2, Wa, ba2)
    return jnp.transpose(out, (0, 2, 1))
